# baseline (speedup 1.0000x reference)
import functools

import jax
import jax.numpy as jnp
from jax.experimental import pallas as pl
from jax.experimental.pallas import tpu as pltpu


def _rel_precompute_kernel(rel_ref, b_ref, bias_ref, out_ref):
    out_ref[...] = (
        jnp.dot(rel_ref[...], b_ref[...], preferred_element_type=jnp.float32)
        + bias_ref[...]
    )


def _edge_kernel(ego_ref, nbr_ref, ids_ref, a_ref, c_ref, rb_ref, w2_ref,
                 b2_ref, out_ref, attn_ref, *, num_rel):
    ids = ids_ref[0, 0, :]
    oh = (ids[:, None] == jax.lax.broadcasted_iota(
        jnp.int32, (ids.shape[0], num_rel), 1)).astype(jnp.float32)
    t = jnp.dot(ego_ref[...], a_ref[...], preferred_element_type=jnp.float32)
    t = t + jnp.dot(nbr_ref[...], c_ref[...], preferred_element_type=jnp.float32)
    t = t + jnp.dot(oh, rb_ref[...], preferred_element_type=jnp.float32)
    out_ref[...] = t
    s = jnp.sum(t * w2_ref[...], axis=1, keepdims=True) + b2_ref[...]
    attn_ref[...] = jnp.where(s >= 0.0, s, 0.2 * s)


def kernel(ego_emb, neighbor_emb, relation_ids, rel_table, W1_w, W1_b, W2_w, W2_b):
    E, D = ego_emb.shape
    R = rel_table.shape[0]
    BK = 5000
    nb = E // BK

    Wt = W1_w.T
    A = Wt[:D]
    B = Wt[D:2 * D]
    C = Wt[2 * D:]

    rb = pl.pallas_call(
        _rel_precompute_kernel,
        out_shape=jax.ShapeDtypeStruct((R, D), jnp.float32),
    )(rel_table, B, W1_b.reshape(1, D))

    ids3 = relation_ids.astype(jnp.int32).reshape(nb, 1, BK)

    out, attn = pl.pallas_call(
        functools.partial(_edge_kernel, num_rel=R),
        grid=(nb,),
        in_specs=[
            pl.BlockSpec((BK, D), lambda i: (i, 0)),
            pl.BlockSpec((BK, D), lambda i: (i, 0)),
            pl.BlockSpec((1, 1, BK), lambda i: (i, 0, 0)),
            pl.BlockSpec((D, D), lambda i: (0, 0)),
            pl.BlockSpec((D, D), lambda i: (0, 0)),
            pl.BlockSpec((R, D), lambda i: (0, 0)),
            pl.BlockSpec((1, D), lambda i: (0, 0)),
            pl.BlockSpec((1, 1), lambda i: (0, 0)),
        ],
        out_specs=[
            pl.BlockSpec((BK, D), lambda i: (i, 0)),
            pl.BlockSpec((BK, 1), lambda i: (i, 0)),
        ],
        out_shape=[
            jax.ShapeDtypeStruct((E, D), jnp.float32),
            jax.ShapeDtypeStruct((E, 1), jnp.float32),
        ],
        compiler_params=pltpu.CompilerParams(
            dimension_semantics=("parallel",)),
    )(ego_emb, neighbor_emb, ids3, A, C, rb, W2_w, W2_b.reshape(1, 1))

    return (out, attn)


# speedup vs baseline: 1.0045x; 1.0045x over previous
import functools

import jax
import jax.numpy as jnp
from jax.experimental import pallas as pl
from jax.experimental.pallas import tpu as pltpu


def _edge_kernel(ego_ref, nbr_ref, ids_ref, a_ref, c_ref, rel_ref, b_ref,
                 b1_ref, w2_ref, b2_ref, out_ref, attn_ref, rb_ref, *,
                 num_rel):
    @pl.when(pl.program_id(0) == 0)
    def _():
        rb_ref[...] = (
            jnp.dot(rel_ref[...], b_ref[...],
                    preferred_element_type=jnp.float32)
            + b1_ref[...]
        )

    ids = ids_ref[0, 0, :]
    oh = (ids[:, None] == jax.lax.broadcasted_iota(
        jnp.int32, (ids.shape[0], num_rel), 1)).astype(jnp.float32)
    t = jnp.dot(ego_ref[...], a_ref[...], preferred_element_type=jnp.float32)
    t = t + jnp.dot(nbr_ref[...], c_ref[...], preferred_element_type=jnp.float32)
    t = t + jnp.dot(oh, rb_ref[...], preferred_element_type=jnp.float32)
    out_ref[...] = t
    s = jnp.sum(t * w2_ref[...], axis=1, keepdims=True) + b2_ref[...]
    attn_ref[...] = jnp.where(s >= 0.0, s, 0.2 * s)


def kernel(ego_emb, neighbor_emb, relation_ids, rel_table, W1_w, W1_b, W2_w, W2_b):
    E, D = ego_emb.shape
    R = rel_table.shape[0]
    BK = 5000
    nb = E // BK

    Wt = W1_w.T
    A = Wt[:D]
    B = Wt[D:2 * D]
    C = Wt[2 * D:]

    ids3 = relation_ids.astype(jnp.int32).reshape(nb, 1, BK)

    out, attn = pl.pallas_call(
        functools.partial(_edge_kernel, num_rel=R),
        grid=(nb,),
        in_specs=[
            pl.BlockSpec((BK, D), lambda i: (i, 0)),
            pl.BlockSpec((BK, D), lambda i: (i, 0)),
            pl.BlockSpec((1, 1, BK), lambda i: (i, 0, 0)),
            pl.BlockSpec((D, D), lambda i: (0, 0)),
            pl.BlockSpec((D, D), lambda i: (0, 0)),
            pl.BlockSpec((R, D), lambda i: (0, 0)),
            pl.BlockSpec((D, D), lambda i: (0, 0)),
            pl.BlockSpec((1, D), lambda i: (0, 0)),
            pl.BlockSpec((1, D), lambda i: (0, 0)),
            pl.BlockSpec((1, 1), lambda i: (0, 0)),
        ],
        out_specs=[
            pl.BlockSpec((BK, D), lambda i: (i, 0)),
            pl.BlockSpec((BK, 1), lambda i: (i, 0)),
        ],
        out_shape=[
            jax.ShapeDtypeStruct((E, D), jnp.float32),
            jax.ShapeDtypeStruct((E, 1), jnp.float32),
        ],
        scratch_shapes=[pltpu.VMEM((R, D), jnp.float32)],
    )(ego_emb, neighbor_emb, ids3, A, C, rel_table, B,
      W1_b.reshape(1, D), W2_w, W2_b.reshape(1, 1))

    return (out, attn)


# speedup vs baseline: 1.0052x; 1.0006x over previous
import functools

import jax
import jax.numpy as jnp
from jax.experimental import pallas as pl
from jax.experimental.pallas import tpu as pltpu


def _edge_kernel(ego_ref, nbr_ref, ids_ref, a_ref, c_ref, rel_ref, b_ref,
                 b1_ref, w2_ref, b2_ref, out_ref, attn_ref, rb_ref, *,
                 num_rel):
    @pl.when(pl.program_id(0) == 0)
    def _():
        rb_ref[...] = (
            jnp.dot(rel_ref[...], b_ref[...],
                    preferred_element_type=jnp.float32)
            + b1_ref[...]
        )

    ids = ids_ref[0, 0, :]
    oh = (ids[:, None] == jax.lax.broadcasted_iota(
        jnp.int32, (ids.shape[0], num_rel), 1)).astype(jnp.float32)
    t = jnp.dot(ego_ref[...], a_ref[...], preferred_element_type=jnp.float32)
    t = t + jnp.dot(nbr_ref[...], c_ref[...], preferred_element_type=jnp.float32)
    t = t + jnp.dot(oh, rb_ref[...], preferred_element_type=jnp.float32)
    out_ref[...] = t
    s = jnp.sum(t * w2_ref[...], axis=1, keepdims=True) + b2_ref[...]
    attn_ref[...] = jnp.where(s >= 0.0, s, 0.2 * s)


def kernel(ego_emb, neighbor_emb, relation_ids, rel_table, W1_w, W1_b, W2_w, W2_b):
    E, D = ego_emb.shape
    R = rel_table.shape[0]
    BK = 6400
    nb = E // BK

    Wt = W1_w.T
    A = Wt[:D]
    B = Wt[D:2 * D]
    C = Wt[2 * D:]

    ids3 = relation_ids.astype(jnp.int32).reshape(nb, 1, BK)

    out, attn = pl.pallas_call(
        functools.partial(_edge_kernel, num_rel=R),
        grid=(nb,),
        in_specs=[
            pl.BlockSpec((BK, D), lambda i: (i, 0)),
            pl.BlockSpec((BK, D), lambda i: (i, 0)),
            pl.BlockSpec((1, 1, BK), lambda i: (i, 0, 0)),
            pl.BlockSpec((D, D), lambda i: (0, 0)),
            pl.BlockSpec((D, D), lambda i: (0, 0)),
            pl.BlockSpec((R, D), lambda i: (0, 0)),
            pl.BlockSpec((D, D), lambda i: (0, 0)),
            pl.BlockSpec((1, D), lambda i: (0, 0)),
            pl.BlockSpec((1, D), lambda i: (0, 0)),
            pl.BlockSpec((1, 1), lambda i: (0, 0)),
        ],
        out_specs=[
            pl.BlockSpec((BK, D), lambda i: (i, 0)),
            pl.BlockSpec((BK, 1), lambda i: (i, 0)),
        ],
        out_shape=[
            jax.ShapeDtypeStruct((E, D), jnp.float32),
            jax.ShapeDtypeStruct((E, 1), jnp.float32),
        ],
        scratch_shapes=[pltpu.VMEM((R, D), jnp.float32)],
    )(ego_emb, neighbor_emb, ids3, A, C, rel_table, B,
      W1_b.reshape(1, D), W2_w, W2_b.reshape(1, 1))

    return (out, attn)
